# baseline (speedup 1.0000x reference)
import functools

import jax
import jax.numpy as jnp
from jax import lax
from jax.experimental import pallas as pl
from jax.experimental.pallas import tpu as pltpu
from jax.experimental.pallas import tpu_sc as plsc

_NC = 2
_NS = 16
_NW = _NC * _NS
_CHUNK = 128


@functools.lru_cache(maxsize=None)
def _make_gather(num_rows: int, dim: int):
    assert num_rows % (_NW * _CHUNK) == 0
    rows_per_w = num_rows // _NW
    cpw = rows_per_w // _CHUNK

    mesh = plsc.VectorSubcoreMesh(core_axis_name="c", subcore_axis_name="s")

    @functools.partial(
        pl.kernel,
        mesh=mesh,
        out_type=jax.ShapeDtypeStruct((num_rows, dim), jnp.float32),
        scratch_types=[
            pltpu.VMEM((cpw, _CHUNK), jnp.int32),
            pltpu.VMEM((_CHUNK, dim), jnp.float32),
            pltpu.SemaphoreType.DMA,
        ],
        compiler_params=pltpu.CompilerParams(use_tc_tiling_on_sc=False),
    )
    def gather(weight_hbm, idx_hbm, out_hbm, idx_v, rows_v, gsem):
        c = lax.axis_index("c")
        s = lax.axis_index("s")
        wid = s * _NC + c
        pltpu.sync_copy(idx_hbm.at[pl.ds(wid * cpw, cpw)], idx_v)

        def chunk(j, carry):
            pltpu.async_copy(weight_hbm.at[idx_v.at[j]], rows_v, gsem).wait()
            pltpu.sync_copy(
                rows_v, out_hbm.at[pl.ds((wid * cpw + j) * _CHUNK, _CHUNK)]
            )
            return carry

        lax.fori_loop(0, cpw, chunk, 0)

    return gather


def kernel(input_, weight):
    batch, hist = input_.shape
    num_rows = batch * hist
    dim = weight.shape[1]
    idx = input_.reshape(num_rows // _CHUNK, _CHUNK).astype(jnp.int32)
    out = _make_gather(num_rows, dim)(weight, idx)
    return out.reshape(batch, hist, dim)


# speedup vs baseline: 1.0652x; 1.0652x over previous
import functools

import jax
import jax.numpy as jnp
from jax import lax
from jax.experimental import pallas as pl
from jax.experimental.pallas import tpu as pltpu
from jax.experimental.pallas import tpu_sc as plsc

_NC = 2
_NS = 16
_NW = _NC * _NS
_CHUNK = 128
_NBUF = 4
_AHEAD = _NBUF - 1


@functools.lru_cache(maxsize=None)
def _make_gather(num_rows: int, dim: int):
    assert num_rows % (_NW * _CHUNK) == 0
    rows_per_w = num_rows // _NW
    cpw = rows_per_w // _CHUNK
    assert cpw > _NBUF

    mesh = plsc.VectorSubcoreMesh(core_axis_name="c", subcore_axis_name="s")

    @functools.partial(
        pl.kernel,
        mesh=mesh,
        out_type=jax.ShapeDtypeStruct((num_rows, dim), jnp.float32),
        scratch_types=[
            pltpu.VMEM((cpw, _CHUNK), jnp.int32),
            pltpu.VMEM((_NBUF, _CHUNK, dim), jnp.float32),
            pltpu.SemaphoreType.DMA,
            pltpu.SemaphoreType.DMA,
        ],
        compiler_params=pltpu.CompilerParams(use_tc_tiling_on_sc=False),
    )
    def gather(weight_hbm, idx_hbm, out_hbm, idx_v, rows_v, gsem, wsem):
        c = lax.axis_index("c")
        s = lax.axis_index("s")
        wid = s * _NC + c
        out_base = wid * rows_per_w
        pltpu.sync_copy(idx_hbm.at[pl.ds(wid * cpw, cpw)], idx_v)

        for b in range(_AHEAD):
            pltpu.async_copy(weight_hbm.at[idx_v.at[b]], rows_v.at[b], gsem)

        def body(j, carry):
            b = lax.rem(j, _NBUF)
            jf = j + _AHEAD

            @pl.when(jf < cpw)
            def _():
                @pl.when(j >= 1)
                def _():
                    bp = lax.rem(j - 1, _NBUF)
                    pltpu.make_async_copy(
                        rows_v.at[bp],
                        out_hbm.at[pl.ds(out_base + (j - 1) * _CHUNK, _CHUNK)],
                        wsem,
                    ).wait()

                pltpu.async_copy(
                    weight_hbm.at[idx_v.at[jf]],
                    rows_v.at[lax.rem(jf, _NBUF)],
                    gsem,
                )

            pltpu.make_async_copy(
                weight_hbm.at[idx_v.at[j]], rows_v.at[b], gsem
            ).wait()
            pltpu.async_copy(
                rows_v.at[b],
                out_hbm.at[pl.ds(out_base + j * _CHUNK, _CHUNK)],
                wsem,
            )
            return carry

        lax.fori_loop(0, cpw, body, 0)

        for i in range(_NBUF):
            j = cpw - _NBUF + i
            pltpu.make_async_copy(
                rows_v.at[j % _NBUF],
                out_hbm.at[pl.ds(out_base + j * _CHUNK, _CHUNK)],
                wsem,
            ).wait()

    return gather


def kernel(input_, weight):
    batch, hist = input_.shape
    num_rows = batch * hist
    dim = weight.shape[1]
    idx = input_.reshape(num_rows // _CHUNK, _CHUNK).astype(jnp.int32)
    out = _make_gather(num_rows, dim)(weight, idx)
    return out.reshape(batch, hist, dim)
